# baseline (speedup 1.0000x reference)
import jax
import jax.numpy as jnp
from jax.experimental import pallas as pl
from jax.experimental.pallas import tpu as pltpu

B, L, H = 16, 2048, 1536
E = 256
C = 3
NC = 26
NCHUNK = 8


def _body(hid_ref, pos_ref, ids_ref, msk_ref, table_ref, w_ref, bias_ref,
          out_ref, fused_ref, rows_ref, copy_sem, row_sem):
    step = B // NCHUNK
    copies = []
    for k in range(NCHUNK):
        cp = pltpu.make_async_copy(
            hid_ref.at[pl.ds(k * step, step)],
            out_ref.at[pl.ds(k * step, step)],
            copy_sem.at[k],
        )
        cp.start()
        copies.append(cp)

    gathers = []
    for i in range(B):
        p = pos_ref[i]
        cp = pltpu.make_async_copy(hid_ref.at[i, p], rows_ref.at[i], row_sem)
        cp.start()
        gathers.append(cp)

    ids = ids_ref[...]
    msk = msk_ref[...]
    denom = jnp.maximum(jnp.sum(msk, axis=1, keepdims=True), 1.0)
    wcoef = msk / denom
    iota_n = jax.lax.broadcasted_iota(jnp.int32, (B, NC), 1)
    wsel = jnp.zeros((B, NC), jnp.float32)
    for c in range(C):
        wsel = wsel + jnp.where(ids[:, c:c + 1] == iota_n,
                                wcoef[:, c:c + 1], 0.0)
    mean_emb = jnp.dot(wsel, table_ref[...],
                       preferred_element_type=jnp.float32)
    fused = (jnp.dot(mean_emb, w_ref[...], preferred_element_type=jnp.float32)
             + bias_ref[...])

    for cp in gathers:
        cp.wait()
    rows_ref[...] = rows_ref[...] + fused

    for cp in copies:
        cp.wait()
    scatters = []
    for i in range(B):
        p = pos_ref[i]
        cp = pltpu.make_async_copy(rows_ref.at[i], out_ref.at[i, p], row_sem)
        cp.start()
        scatters.append(cp)
    for cp in scatters:
        cp.wait()
    del fused_ref


def kernel(hidden_states, comp_ids, comp_mask, positions, comp_table, W, b):
    return pl.pallas_call(
        _body,
        out_shape=jax.ShapeDtypeStruct((B, L, H), jnp.float32),
        in_specs=[
            pl.BlockSpec(memory_space=pltpu.HBM),
            pl.BlockSpec(memory_space=pltpu.SMEM),
            pl.BlockSpec(memory_space=pltpu.VMEM),
            pl.BlockSpec(memory_space=pltpu.VMEM),
            pl.BlockSpec(memory_space=pltpu.VMEM),
            pl.BlockSpec(memory_space=pltpu.VMEM),
            pl.BlockSpec(memory_space=pltpu.VMEM),
        ],
        out_specs=pl.BlockSpec(memory_space=pltpu.HBM),
        scratch_shapes=[
            pltpu.VMEM((B, H), jnp.float32),
            pltpu.VMEM((B, H), jnp.float32),
            pltpu.SemaphoreType.DMA((NCHUNK,)),
            pltpu.SemaphoreType.DMA,
        ],
    )(
        hidden_states,
        positions.astype(jnp.int32),
        comp_ids.astype(jnp.int32),
        comp_mask,
        comp_table,
        W,
        b.reshape(1, H),
    )


# speedup vs baseline: 45.6709x; 45.6709x over previous
import jax
import jax.numpy as jnp
from jax.experimental import pallas as pl
from jax.experimental.pallas import tpu as pltpu

B, L, H = 16, 2048, 1536
E = 256
C = 3
NC = 26
TL = 512
NJ = L // TL


def _body(pos_ref, ids_ref, msk_ref, table_ref, w_ref, bias_ref, hid_ref,
          out_ref, fused_ref):
    b = pl.program_id(0)
    j = pl.program_id(1)

    @pl.when(jnp.logical_and(b == 0, j == 0))
    def _():
        ids = ids_ref[...]
        msk = msk_ref[...]
        denom = jnp.maximum(jnp.sum(msk, axis=1, keepdims=True), 1.0)
        wcoef = msk / denom
        iota_n = jax.lax.broadcasted_iota(jnp.int32, (B, NC), 1)
        wsel = jnp.zeros((B, NC), jnp.float32)
        for c in range(C):
            wsel = wsel + jnp.where(ids[:, c:c + 1] == iota_n,
                                    wcoef[:, c:c + 1], 0.0)
        mean_emb = jnp.dot(wsel, table_ref[...],
                           preferred_element_type=jnp.float32)
        fused_ref[...] = (
            jnp.dot(mean_emb, w_ref[...], preferred_element_type=jnp.float32)
            + bias_ref[...])

    out_ref[...] = hid_ref[...]
    p = pos_ref[b]

    @pl.when(p // TL == j)
    def _():
        q = p - j * TL
        out_ref[0, pl.ds(q, 1), :] += fused_ref[pl.ds(b, 1), :]


def kernel(hidden_states, comp_ids, comp_mask, positions, comp_table, W, b):
    grid = (B, NJ)
    return pl.pallas_call(
        _body,
        grid=grid,
        out_shape=jax.ShapeDtypeStruct((B, L, H), jnp.float32),
        in_specs=[
            pl.BlockSpec(memory_space=pltpu.SMEM),
            pl.BlockSpec((B, C), lambda b, j: (0, 0)),
            pl.BlockSpec((B, C), lambda b, j: (0, 0)),
            pl.BlockSpec((NC, E), lambda b, j: (0, 0)),
            pl.BlockSpec((E, H), lambda b, j: (0, 0)),
            pl.BlockSpec((1, H), lambda b, j: (0, 0)),
            pl.BlockSpec((1, TL, H), lambda b, j: (b, j, 0)),
        ],
        out_specs=pl.BlockSpec((1, TL, H), lambda b, j: (b, j, 0)),
        scratch_shapes=[
            pltpu.VMEM((B, H), jnp.float32),
        ],
    )(
        positions.astype(jnp.int32),
        comp_ids.astype(jnp.int32),
        comp_mask,
        comp_table,
        W,
        b.reshape(1, H),
        hidden_states,
    )


# speedup vs baseline: 46.6245x; 1.0209x over previous
import jax
import jax.numpy as jnp
from jax.experimental import pallas as pl
from jax.experimental.pallas import tpu as pltpu

B, L, H = 16, 2048, 1536
E = 256
C = 3
NC = 26

R = B * L
CH = 256
NCH = R // CH
NBUF = 16
LAG = 8


def _body(hid_ref, pos_ref, ids_ref, msk_ref, table_ref, w_ref, bias_ref,
          out_ref, buf_ref, fused_ref, rows_ref, in_sems, out_sems, row_sem):
    def in_cp(k):
        return pltpu.make_async_copy(
            hid_ref.at[pl.ds(k * CH, CH)], buf_ref.at[k % NBUF],
            in_sems.at[k % NBUF])

    def out_cp(k):
        return pltpu.make_async_copy(
            buf_ref.at[k % NBUF], out_ref.at[pl.ds(k * CH, CH)],
            out_sems.at[k % NBUF])

    for k in range(NBUF):
        in_cp(k).start()

    gathers = []
    for i in range(B):
        g = i * L + pos_ref[i]
        cp = pltpu.make_async_copy(hid_ref.at[g], rows_ref.at[i], row_sem)
        cp.start()
        gathers.append(cp)

    ids = ids_ref[...]
    msk = msk_ref[...]
    denom = jnp.maximum(jnp.sum(msk, axis=1, keepdims=True), 1.0)
    wcoef = msk / denom
    iota_n = jax.lax.broadcasted_iota(jnp.int32, (B, NC), 1)
    wsel = jnp.zeros((B, NC), jnp.float32)
    for c in range(C):
        wsel = wsel + jnp.where(ids[:, c:c + 1] == iota_n,
                                wcoef[:, c:c + 1], 0.0)
    mean_emb = jnp.dot(wsel, table_ref[...],
                       preferred_element_type=jnp.float32)
    fused = (jnp.dot(mean_emb, w_ref[...], preferred_element_type=jnp.float32)
             + bias_ref[...])

    for cp in gathers:
        cp.wait()
    rows_ref[...] = rows_ref[...] + fused

    for k in range(NCH):
        if k >= LAG:
            out_cp(k - LAG).wait()
            nxt = k - LAG + NBUF
            if nxt < NCH:
                in_cp(nxt).start()
        in_cp(k).wait()
        out_cp(k).start()
    for k in range(max(0, NCH - LAG), NCH):
        out_cp(k).wait()

    scatters = []
    for i in range(B):
        g = i * L + pos_ref[i]
        cp = pltpu.make_async_copy(rows_ref.at[i], out_ref.at[g], row_sem)
        cp.start()
        scatters.append(cp)
    for cp in scatters:
        cp.wait()
    del fused_ref


def kernel(hidden_states, comp_ids, comp_mask, positions, comp_table, W, b):
    out = pl.pallas_call(
        _body,
        out_shape=jax.ShapeDtypeStruct((R, H), jnp.float32),
        in_specs=[
            pl.BlockSpec(memory_space=pltpu.HBM),
            pl.BlockSpec(memory_space=pltpu.SMEM),
            pl.BlockSpec(memory_space=pltpu.VMEM),
            pl.BlockSpec(memory_space=pltpu.VMEM),
            pl.BlockSpec(memory_space=pltpu.VMEM),
            pl.BlockSpec(memory_space=pltpu.VMEM),
            pl.BlockSpec(memory_space=pltpu.VMEM),
        ],
        out_specs=pl.BlockSpec(memory_space=pltpu.HBM),
        scratch_shapes=[
            pltpu.VMEM((NBUF, CH, H), jnp.float32),
            pltpu.VMEM((B, H), jnp.float32),
            pltpu.VMEM((B, H), jnp.float32),
            pltpu.SemaphoreType.DMA((NBUF,)),
            pltpu.SemaphoreType.DMA((NBUF,)),
            pltpu.SemaphoreType.DMA,
        ],
    )(
        hidden_states.reshape(R, H),
        positions.astype(jnp.int32),
        comp_ids.astype(jnp.int32),
        comp_mask,
        comp_table,
        W,
        b.reshape(1, H),
    )
    return out.reshape(B, L, H)


# speedup vs baseline: 46.6742x; 1.0011x over previous
import jax
import jax.numpy as jnp
from jax.experimental import pallas as pl
from jax.experimental.pallas import tpu as pltpu

B, L, H = 16, 2048, 1536
E = 256
C = 3
NC = 26

R = B * L
CH = 512
NCH = R // CH
NBUF = 8
LAG = 4


def _body(hid_ref, pos_ref, ids_ref, msk_ref, table_ref, w_ref, bias_ref,
          out_ref, buf_ref, fused_ref, rows_ref, in_sems, out_sems, row_sem):
    def in_cp(k):
        return pltpu.make_async_copy(
            hid_ref.at[pl.ds(k * CH, CH)], buf_ref.at[k % NBUF],
            in_sems.at[k % NBUF])

    def out_cp(k):
        return pltpu.make_async_copy(
            buf_ref.at[k % NBUF], out_ref.at[pl.ds(k * CH, CH)],
            out_sems.at[k % NBUF])

    for k in range(NBUF):
        in_cp(k).start()

    gathers = []
    for i in range(B):
        g = i * L + pos_ref[i]
        cp = pltpu.make_async_copy(hid_ref.at[g], rows_ref.at[i], row_sem)
        cp.start()
        gathers.append(cp)

    ids = ids_ref[...]
    msk = msk_ref[...]
    denom = jnp.maximum(jnp.sum(msk, axis=1, keepdims=True), 1.0)
    wcoef = msk / denom
    iota_n = jax.lax.broadcasted_iota(jnp.int32, (B, NC), 1)
    wsel = jnp.zeros((B, NC), jnp.float32)
    for c in range(C):
        wsel = wsel + jnp.where(ids[:, c:c + 1] == iota_n,
                                wcoef[:, c:c + 1], 0.0)
    mean_emb = jnp.dot(wsel, table_ref[...],
                       preferred_element_type=jnp.float32)
    fused = (jnp.dot(mean_emb, w_ref[...], preferred_element_type=jnp.float32)
             + bias_ref[...])

    for cp in gathers:
        cp.wait()
    rows_ref[...] = rows_ref[...] + fused

    for k in range(NCH):
        if k >= LAG:
            out_cp(k - LAG).wait()
            nxt = k - LAG + NBUF
            if nxt < NCH:
                in_cp(nxt).start()
        in_cp(k).wait()
        out_cp(k).start()
    for k in range(max(0, NCH - LAG), NCH):
        out_cp(k).wait()

    scatters = []
    for i in range(B):
        g = i * L + pos_ref[i]
        cp = pltpu.make_async_copy(rows_ref.at[i], out_ref.at[g], row_sem)
        cp.start()
        scatters.append(cp)
    for cp in scatters:
        cp.wait()
    del fused_ref


def kernel(hidden_states, comp_ids, comp_mask, positions, comp_table, W, b):
    out = pl.pallas_call(
        _body,
        out_shape=jax.ShapeDtypeStruct((R, H), jnp.float32),
        in_specs=[
            pl.BlockSpec(memory_space=pltpu.HBM),
            pl.BlockSpec(memory_space=pltpu.SMEM),
            pl.BlockSpec(memory_space=pltpu.VMEM),
            pl.BlockSpec(memory_space=pltpu.VMEM),
            pl.BlockSpec(memory_space=pltpu.VMEM),
            pl.BlockSpec(memory_space=pltpu.VMEM),
            pl.BlockSpec(memory_space=pltpu.VMEM),
        ],
        out_specs=pl.BlockSpec(memory_space=pltpu.HBM),
        scratch_shapes=[
            pltpu.VMEM((NBUF, CH, H), jnp.float32),
            pltpu.VMEM((B, H), jnp.float32),
            pltpu.VMEM((B, H), jnp.float32),
            pltpu.SemaphoreType.DMA((NBUF,)),
            pltpu.SemaphoreType.DMA((NBUF,)),
            pltpu.SemaphoreType.DMA,
        ],
    )(
        hidden_states.reshape(R, H),
        positions.astype(jnp.int32),
        comp_ids.astype(jnp.int32),
        comp_mask,
        comp_table,
        W,
        b.reshape(1, H),
    )
    return out.reshape(B, L, H)
